# Initial kernel scaffold; baseline (speedup 1.0000x reference)
import jax
import jax.numpy as jnp
from jax.experimental import pallas as pl


def kernel(input_one, input_two, W):
    raise NotImplementedError("write your pallas kernel here")


# speedup vs baseline: 9.3394x; 9.3394x over previous
import functools

import jax
import jax.numpy as jnp
from jax import lax
from jax.experimental import pallas as pl
from jax.experimental.pallas import tpu as pltpu
from jax.experimental.pallas import tpu_sc as plsc

_B = 1024
_D = 64
_L = 16

_INFO = plsc.get_sparse_core_info()
_NC = _INFO.num_cores
_NS = _INFO.num_subcores
_NW = _NC * _NS
_BPW = _B // _NW


def _body(i1_hbm, i2_hbm, w_hbm, out_hbm,
          idx1_v, idx2_v, rows1_v, rows2_v, scale_v, sem1, sem2):
    wid = lax.axis_index("s") * _NC + lax.axis_index("c")
    base = wid * _BPW

    pltpu.sync_copy(i1_hbm.at[pl.ds(base, _BPW)], idx1_v)
    pltpu.sync_copy(i2_hbm.at[pl.ds(base, _BPW)], idx2_v)

    c1 = pltpu.async_copy(w_hbm.at[idx1_v], rows1_v, sem1)
    c2 = pltpu.async_copy(w_hbm.at[idx2_v], rows2_v, sem2)

    half = jnp.full((_L,), 0.5, jnp.float32)
    one = jnp.full((_L,), 1.0, jnp.float32)
    for c in range(_BPW // _L):
        sl = pl.ds(c * _L, _L)
        scale_v[sl] = jnp.where(idx1_v[sl] == idx2_v[sl], half, one)

    c1.wait()
    c2.wait()

    for i in range(_BPW):
        s = plsc.load_gather(scale_v, [jnp.full((_L,), i, jnp.int32)])
        for d in range(_D // _L):
            dl = pl.ds(d * _L, _L)
            rows1_v[i, dl] = (rows1_v[i, dl] + rows2_v[i, dl]) * s

    pltpu.sync_copy(rows1_v, out_hbm.at[pl.ds(base, _BPW)])


@functools.partial(
    pl.kernel,
    mesh=plsc.VectorSubcoreMesh(core_axis_name="c", subcore_axis_name="s"),
    out_type=jax.ShapeDtypeStruct((_B, _D), jnp.float32),
    scratch_types=[
        pltpu.VMEM((_BPW,), jnp.int32),
        pltpu.VMEM((_BPW,), jnp.int32),
        pltpu.VMEM((_BPW, _D), jnp.float32),
        pltpu.VMEM((_BPW, _D), jnp.float32),
        pltpu.VMEM((_BPW,), jnp.float32),
        pltpu.SemaphoreType.DMA,
        pltpu.SemaphoreType.DMA,
    ],
    compiler_params=pltpu.CompilerParams(
        needs_layout_passes=False, use_tc_tiling_on_sc=False),
)
def _two_hot_sc(i1_hbm, i2_hbm, w_hbm, out_hbm, *scratch):
    _body(i1_hbm, i2_hbm, w_hbm, out_hbm, *scratch)


def kernel(input_one, input_two, W):
    return _two_hot_sc(input_one.astype(jnp.int32),
                       input_two.astype(jnp.int32), W)
